# Initial kernel scaffold; baseline (speedup 1.0000x reference)
import jax
import jax.numpy as jnp
from jax.experimental import pallas as pl


def kernel(item_id, emb_item):
    raise NotImplementedError("write your pallas kernel here")


# speedup vs baseline: 1.6817x; 1.6817x over previous
import functools

import jax
import jax.numpy as jnp
from jax import lax
from jax.experimental import pallas as pl
from jax.experimental.pallas import tpu as pltpu
from jax.experimental.pallas import tpu_sc as plsc

EMB_DIM = 64
K = 128


@functools.partial(jax.jit, static_argnames=("num_rows",))
def _gather_rows(emb_item, idx2d, *, num_rows):
    info = plsc.get_sparse_core_info()
    nc, ns = info.num_cores, info.num_subcores
    nw = nc * ns
    steps_per_w = idx2d.shape[0] // nw

    mesh = plsc.VectorSubcoreMesh(core_axis_name="c", subcore_axis_name="s")

    @functools.partial(
        pl.kernel,
        mesh=mesh,
        out_type=jax.ShapeDtypeStruct((num_rows, EMB_DIM), jnp.float32),
        scratch_types=[
            pltpu.VMEM((steps_per_w, K), jnp.int32),
            pltpu.VMEM((K, EMB_DIM), jnp.float32),
            pltpu.SemaphoreType.DMA,
        ],
        compiler_params=pltpu.CompilerParams(use_tc_tiling_on_sc=False),
    )
    def k(table_hbm, idx_hbm, out_hbm, idx_v, rows_v, sem):
        wid = lax.axis_index("s") * nc + lax.axis_index("c")
        pltpu.sync_copy(idx_hbm.at[pl.ds(wid * steps_per_w, steps_per_w)], idx_v)
        base_row = wid * (steps_per_w * K)

        @pl.loop(0, steps_per_w)
        def _(j):
            pltpu.async_copy(table_hbm.at[idx_v.at[j]], rows_v, sem).wait()
            pltpu.sync_copy(rows_v, out_hbm.at[pl.ds(base_row + j * K, K)])

    return k(emb_item, idx2d)


def kernel(item_id, emb_item):
    batch, hist = item_id.shape
    num_rows = batch * hist
    idx2d = item_id.astype(jnp.int32).reshape(num_rows // K, K)
    out = _gather_rows(emb_item, idx2d, num_rows=num_rows)
    return out.reshape(batch, hist, EMB_DIM)


# speedup vs baseline: 1.8713x; 1.1127x over previous
import functools

import jax
import jax.numpy as jnp
from jax import lax
from jax.experimental import pallas as pl
from jax.experimental.pallas import tpu as pltpu
from jax.experimental.pallas import tpu_sc as plsc

EMB_DIM = 64
K = 128
NBUF = 8


@functools.partial(jax.jit, static_argnames=("num_rows",))
def _gather_rows(emb_item, idx2d, *, num_rows):
    info = plsc.get_sparse_core_info()
    nc, ns = info.num_cores, info.num_subcores
    nw = nc * ns
    steps_per_w = idx2d.shape[0] // nw
    n_outer = steps_per_w // NBUF

    mesh = plsc.VectorSubcoreMesh(core_axis_name="c", subcore_axis_name="s")

    @functools.partial(
        pl.kernel,
        mesh=mesh,
        out_type=jax.ShapeDtypeStruct((num_rows, EMB_DIM), jnp.float32),
        scratch_types=[
            pltpu.VMEM((steps_per_w, K), jnp.int32),
            pltpu.VMEM((NBUF, K, EMB_DIM), jnp.float32),
            pltpu.SemaphoreType.DMA((NBUF,)),
            pltpu.SemaphoreType.DMA((NBUF,)),
        ],
        compiler_params=pltpu.CompilerParams(use_tc_tiling_on_sc=False),
    )
    def k(table_hbm, idx_hbm, out_hbm, idx_v, rows_v, gsem, osem):
        wid = lax.axis_index("s") * nc + lax.axis_index("c")
        pltpu.sync_copy(idx_hbm.at[pl.ds(wid * steps_per_w, steps_per_w)], idx_v)
        base_row = wid * (steps_per_w * K)

        def fire_gather(b, j):
            pltpu.async_copy(table_hbm.at[idx_v.at[j]], rows_v.at[b], gsem.at[b])

        def wait_gather(b, j):
            pltpu.make_async_copy(
                table_hbm.at[idx_v.at[j]], rows_v.at[b], gsem.at[b]
            ).wait()

        def out_slice(j):
            return out_hbm.at[pl.ds(base_row + j * K, K)]

        def fire_out(b, j):
            pltpu.async_copy(rows_v.at[b], out_slice(j), osem.at[b])

        def wait_out(b, j):
            pltpu.make_async_copy(rows_v.at[b], out_slice(j), osem.at[b]).wait()

        for b in range(NBUF):
            fire_gather(b, b)

        @pl.loop(0, n_outer)
        def _(g):
            j0 = g * NBUF
            for b in range(NBUF):
                wait_gather(b, j0 + b)
                fire_out(b, j0 + b)

            @pl.when(g < n_outer - 1)
            def _():
                for b in range(NBUF):
                    wait_out(b, j0 + b)
                    fire_gather(b, j0 + b + NBUF)

        for b in range(NBUF):
            wait_out(b, (n_outer - 1) * NBUF + b)

    return k(emb_item, idx2d)


def kernel(item_id, emb_item):
    batch, hist = item_id.shape
    num_rows = batch * hist
    idx2d = item_id.astype(jnp.int32).reshape(num_rows // K, K)
    out = _gather_rows(emb_item, idx2d, num_rows=num_rows)
    return out.reshape(batch, hist, EMB_DIM)
